# Initial kernel scaffold; baseline (speedup 1.0000x reference)
import jax
import jax.numpy as jnp
from jax.experimental import pallas as pl


def kernel(u_pos, v_pos, book_pos, v_neg_city, v_neg_country, cand_embed, contx_embed):
    raise NotImplementedError("write your pallas kernel here")


# speedup vs baseline: 1.4769x; 1.4769x over previous
import functools

import jax
import jax.numpy as jnp
from jax import lax
from jax.experimental import pallas as pl
from jax.experimental.pallas import tpu as pltpu
from jax.experimental.pallas import tpu_sc as plsc

_B = 16384
_D = 32
_L = 20
_NNEG = 50
_R = 128
_NW = 32
_PER_W = _B // _NW
_E = 8
_CHUNKS = _PER_W // _E


def _sc_scores(cand_hbm, ctx_hbm, u_pos_hbm, ctx_idx_hbm, out_hbm,
               u_idx_v, idx_v, u_rows_v, rows_v, scores_v, sem):
    wid = lax.axis_index("s") * 2 + lax.axis_index("c")
    lane = lax.iota(jnp.int32, 16)

    def chunk_body(c, _):
        base = wid * _PER_W + c * _E
        pltpu.sync_copy(u_pos_hbm.at[pl.ds(base, _E)], u_idx_v)
        pltpu.sync_copy(ctx_idx_hbm.at[pl.ds(base, _E)], idx_v)
        cp_u = pltpu.async_copy(cand_hbm.at[u_idx_v], u_rows_v, sem)
        cps = [
            pltpu.async_copy(ctx_hbm.at[idx_v.at[e]],
                             rows_v.at[pl.ds(e * _R, _R)], sem)
            for e in range(_E)
        ]
        cp_u.wait()
        for cp in cps:
            cp.wait()

        for e in range(_E):
            rowids = [jnp.full((16,), e * _R + g * 16, jnp.int32) + lane
                      for g in range(8)]
            e_splat = jnp.full((16,), e, jnp.int32)

            def d_body(d, accs):
                d_splat = jnp.full((16,), d, jnp.int32)
                u_vec = plsc.load_gather(u_rows_v, [e_splat, d_splat])
                new = []
                for g in range(8):
                    v = plsc.load_gather(rows_v, [rowids[g], d_splat])
                    new.append(accs[g] + v * u_vec)
                return tuple(new)

            accs = lax.fori_loop(
                0, _D, d_body,
                tuple(jnp.zeros((16,), jnp.float32) for _ in range(8)))
            for g in range(8):
                scores_v[e, pl.ds(g * 16, 16)] = accs[g]

        pltpu.sync_copy(scores_v, out_hbm.at[pl.ds(base, _E)])
        return 0

    lax.fori_loop(0, _CHUNKS, chunk_body, 0)


def _sc_call(cand_embed, contx_embed, u_pos, ctx_idx):
    mesh = plsc.VectorSubcoreMesh(core_axis_name="c", subcore_axis_name="s")
    kfn = functools.partial(
        pl.kernel,
        mesh=mesh,
        out_type=jax.ShapeDtypeStruct((_B, _R), jnp.float32),
        scratch_types=[
            pltpu.VMEM((_E,), jnp.int32),
            pltpu.VMEM((_E, _R), jnp.int32),
            pltpu.VMEM((_E, _D), jnp.float32),
            pltpu.VMEM((_E * _R, _D), jnp.float32),
            pltpu.VMEM((_E, _R), jnp.float32),
            pltpu.SemaphoreType.DMA,
        ],
        compiler_params=pltpu.CompilerParams(
            needs_layout_passes=False, use_tc_tiling_on_sc=False),
    )(_sc_scores)
    return kfn(cand_embed, contx_embed, u_pos, ctx_idx)


def _tc_loss_body(scores_ref, vpos_ref, out_ref):
    s = scores_ref[...]
    vp = vpos_ref[...]
    mask = (vp != 0).astype(jnp.float32)

    def logsig(x):
        return jnp.minimum(x, 0.0) - jnp.log1p(jnp.exp(-jnp.abs(x)))

    s_pos = jnp.sum(s[:, :_L] * mask, axis=1)
    s_book = s[:, _L]
    neg = s[:, _L + 1:_L + 1 + 2 * _NNEG]
    loss = -(logsig(s_pos) + logsig(s_book)
             + jnp.sum(logsig(-neg), axis=1))
    out_ref[...] = loss


def _tc_loss(scores, v_pos):
    bs = 2048
    return pl.pallas_call(
        _tc_loss_body,
        grid=(_B // bs,),
        in_specs=[
            pl.BlockSpec((bs, _R), lambda i: (i, 0)),
            pl.BlockSpec((bs, _L), lambda i: (i, 0)),
        ],
        out_specs=pl.BlockSpec((bs,), lambda i: (i,)),
        out_shape=jax.ShapeDtypeStruct((_B,), jnp.float32),
    )(scores, v_pos)


def kernel(u_pos, v_pos, book_pos, v_neg_city, v_neg_country,
           cand_embed, contx_embed):
    ctx_idx = jnp.concatenate(
        [v_pos, book_pos[:, None], v_neg_city, v_neg_country,
         jnp.zeros((_B, _R - (_L + 1 + 2 * _NNEG)), jnp.int32)], axis=1)
    scores = _sc_call(cand_embed, contx_embed, u_pos, ctx_idx)
    return _tc_loss(scores, v_pos)
